# baseline (speedup 1.0000x reference)
import functools

import jax
import jax.numpy as jnp
from jax import lax
from jax.experimental import pallas as pl
from jax.experimental.pallas import tpu as pltpu
from jax.experimental.pallas import tpu_sc as plsc

N_ITEMS = 100000
D = 128
N = 50
NPOS_PAD = 64
BATCH = 4096

NC = 2
NS = 16
NW = NC * NS
ROWS = BATCH * N
ROWS_PER_W = ROWS // NW
CHUNK = 256
GSTREAM = 128
NCH = ROWS_PER_W // CHUNK
NBUF = 3
LANES = 16


def _build_kernel():
    mesh = plsc.VectorSubcoreMesh(core_axis_name="c", subcore_axis_name="s")

    @functools.partial(
        pl.kernel,
        mesh=mesh,
        out_type=jax.ShapeDtypeStruct((ROWS, D), jnp.float32),
        scratch_types=(
            [pltpu.VMEM((ROWS_PER_W,), jnp.int32),
             pltpu.VMEM((16, D), jnp.float32),
             pltpu.VMEM((NBUF, CHUNK, D), jnp.float32)]
            + [pltpu.SemaphoreType.DMA] * (2 * NBUF)
        ),
    )
    def emb_kernel(x_hbm, table_hbm, pos_hbm, out_hbm, idx_v, pos_v, buf,
                   *sems):
        gsem = sems[:NBUF]
        ssem = sems[NBUF:]
        wid = lax.axis_index("s") * NC + lax.axis_index("c")
        row0 = wid * ROWS_PER_W
        n0 = pl.multiple_of(((NCH * wid) // (BATCH // CHUNK)) & ~7, 8)

        pltpu.sync_copy(x_hbm.at[wid], idx_v)
        pltpu.sync_copy(pos_hbm.at[pl.ds(n0, 16)], pos_v)

        def g_issue(cg, b):
            for s in range(CHUNK // GSTREAM):
                pltpu.async_copy(
                    table_hbm.at[idx_v.at[pl.ds(cg * CHUNK + s * GSTREAM,
                                                GSTREAM)]],
                    buf.at[b, pl.ds(s * GSTREAM, GSTREAM)], gsem[b])

        def g_wait(cg, b):
            for s in range(CHUNK // GSTREAM):
                pltpu.make_async_copy(
                    table_hbm.at[idx_v.at[pl.ds(cg * CHUNK + s * GSTREAM,
                                                GSTREAM)]],
                    buf.at[b, pl.ds(s * GSTREAM, GSTREAM)], gsem[b]).wait()

        def s_issue(cg, b):
            pltpu.async_copy(buf.at[b],
                             out_hbm.at[pl.ds(row0 + cg * CHUNK, CHUNK)],
                             ssem[b])

        def s_wait(b):
            pltpu.make_async_copy(buf.at[b], out_hbm.at[pl.ds(0, CHUNK)],
                                  ssem[b]).wait()

        def add_pos(cg, b):
            n_loc = (NCH * wid + cg) // (BATCH // CHUNK) - n0
            ps = [pos_v[n_loc, pl.ds(c * LANES, LANES)]
                  for c in range(D // LANES)]

            def row(r, carry):
                for c in range(D // LANES):
                    sl = pl.ds(c * LANES, LANES)
                    buf[b, r, sl] = buf[b, r, sl] + ps[c]
                return carry

            lax.fori_loop(0, CHUNK, row, 0)

        def process(cg, b):
            bn = (b + 1) % NBUF
            static = isinstance(cg, int)
            if not static or cg >= 2:
                s_wait(bn)
            if not static or cg + 1 < NCH:
                g_issue(cg + 1, bn)
            g_wait(cg, b)
            add_pos(cg, b)
            s_issue(cg, b)

        g_issue(0, 0)
        for b in range(NBUF):
            process(b, b)

        def round_body(r, carry):
            cg0 = r * NBUF
            for b in range(NBUF):
                process(cg0 + b, b)
            return carry

        lax.fori_loop(1, (NCH - 1) // NBUF, round_body, 0)

        process(NCH - 1, (NCH - 1) % NBUF)
        s_wait((NCH - 2) % NBUF)
        s_wait((NCH - 1) % NBUF)

    return emb_kernel


_EMB_KERNEL = _build_kernel()


def kernel(X, embedding, positional_embedding):
    Xt = jnp.transpose(X.astype(jnp.int32)).reshape(NW, ROWS_PER_W)
    pos_pad = jnp.pad(positional_embedding, ((0, NPOS_PAD - N), (0, 0)))
    out = _EMB_KERNEL(Xt, embedding, pos_pad)
    return jnp.transpose(out.reshape(N, BATCH, D), (1, 0, 2))


# speedup vs baseline: 1.0060x; 1.0060x over previous
import functools

import jax
import jax.numpy as jnp
from jax import lax
from jax.experimental import pallas as pl
from jax.experimental.pallas import tpu as pltpu
from jax.experimental.pallas import tpu_sc as plsc

N_ITEMS = 100000
D = 128
N = 50
NPOS_PAD = 64
BATCH = 4096

NC = 2
NS = 16
NW = NC * NS
ROWS = BATCH * N
ROWS_PER_W = ROWS // NW
CHUNK = 256
GSTREAM = 128
NCH = ROWS_PER_W // CHUNK
NBUF = 3
LANES = 16


def _build_kernel():
    mesh = plsc.VectorSubcoreMesh(core_axis_name="c", subcore_axis_name="s")

    @functools.partial(
        pl.kernel,
        mesh=mesh,
        compiler_params=pltpu.CompilerParams(skip_device_barrier=True),
        out_type=jax.ShapeDtypeStruct((ROWS, D), jnp.float32),
        scratch_types=(
            [pltpu.VMEM((ROWS_PER_W,), jnp.int32),
             pltpu.VMEM((16, D), jnp.float32),
             pltpu.VMEM((NBUF, CHUNK, D), jnp.float32)]
            + [pltpu.SemaphoreType.DMA] * (2 * NBUF)
        ),
    )
    def emb_kernel(x_hbm, table_hbm, pos_hbm, out_hbm, idx_v, pos_v, buf,
                   *sems):
        gsem = sems[:NBUF]
        ssem = sems[NBUF:]
        wid = lax.axis_index("s") * NC + lax.axis_index("c")
        row0 = wid * ROWS_PER_W
        n0 = pl.multiple_of(((NCH * wid) // (BATCH // CHUNK)) & ~7, 8)

        pltpu.sync_copy(x_hbm.at[wid], idx_v)
        pltpu.sync_copy(pos_hbm.at[pl.ds(n0, 16)], pos_v)

        def g_issue(cg, b):
            for s in range(CHUNK // GSTREAM):
                pltpu.async_copy(
                    table_hbm.at[idx_v.at[pl.ds(cg * CHUNK + s * GSTREAM,
                                                GSTREAM)]],
                    buf.at[b, pl.ds(s * GSTREAM, GSTREAM)], gsem[b])

        def g_wait(cg, b):
            for s in range(CHUNK // GSTREAM):
                pltpu.make_async_copy(
                    table_hbm.at[idx_v.at[pl.ds(cg * CHUNK + s * GSTREAM,
                                                GSTREAM)]],
                    buf.at[b, pl.ds(s * GSTREAM, GSTREAM)], gsem[b]).wait()

        def s_issue(cg, b):
            pltpu.async_copy(buf.at[b],
                             out_hbm.at[pl.ds(row0 + cg * CHUNK, CHUNK)],
                             ssem[b])

        def s_wait(b):
            pltpu.make_async_copy(buf.at[b], out_hbm.at[pl.ds(0, CHUNK)],
                                  ssem[b]).wait()

        def add_pos(cg, b):
            n_loc = (NCH * wid + cg) // (BATCH // CHUNK) - n0
            ps = [pos_v[n_loc, pl.ds(c * LANES, LANES)]
                  for c in range(D // LANES)]

            def row(r, carry):
                for c in range(D // LANES):
                    sl = pl.ds(c * LANES, LANES)
                    buf[b, r, sl] = buf[b, r, sl] + ps[c]
                return carry

            lax.fori_loop(0, CHUNK, row, 0)

        def process(cg, b):
            bn = (b + 1) % NBUF
            static = isinstance(cg, int)
            if not static or cg >= 2:
                s_wait(bn)
            if not static or cg + 1 < NCH:
                g_issue(cg + 1, bn)
            g_wait(cg, b)
            add_pos(cg, b)
            s_issue(cg, b)

        g_issue(0, 0)
        for b in range(NBUF):
            process(b, b)

        def round_body(r, carry):
            cg0 = r * NBUF
            for b in range(NBUF):
                process(cg0 + b, b)
            return carry

        lax.fori_loop(1, (NCH - 1) // NBUF, round_body, 0)

        process(NCH - 1, (NCH - 1) % NBUF)
        s_wait((NCH - 2) % NBUF)
        s_wait((NCH - 1) % NBUF)

    return emb_kernel


_EMB_KERNEL = _build_kernel()


def kernel(X, embedding, positional_embedding):
    Xt = jnp.transpose(X.astype(jnp.int32)).reshape(NW, ROWS_PER_W)
    pos_pad = jnp.pad(positional_embedding, ((0, NPOS_PAD - N), (0, 0)))
    out = _EMB_KERNEL(Xt, embedding, pos_pad)
    return jnp.transpose(out.reshape(N, BATCH, D), (1, 0, 2))
